# Initial kernel scaffold; baseline (speedup 1.0000x reference)
import jax
import jax.numpy as jnp
from jax.experimental import pallas as pl


def kernel(input_ids, input_pos, vectors, word_table, pos_table, ln_gamma, ln_beta):
    raise NotImplementedError("write your pallas kernel here")


# speedup vs baseline: 1.2876x; 1.2876x over previous
import jax
import jax.numpy as jnp
from jax.experimental import pallas as pl
from jax.experimental.pallas import tpu as pltpu
from jax.experimental.pallas import tpu_sc as plsc

B, L, H, V = 16, 2048, 1024, 32000
PAD = 0
EPS = 1e-12

N_TOK = B * L
NC, NS = 2, 16
NW = NC * NS
B_PER_W = N_TOK // NW
CH = 32
NCH = B_PER_W // CH
BL = 256


def _sc_gather(word_table, flat_ids):
    mesh = plsc.VectorSubcoreMesh(core_axis_name="c", subcore_axis_name="s")

    @pl.kernel(out_type=jax.ShapeDtypeStruct((N_TOK, H), word_table.dtype),
               mesh=mesh,
               scratch_types=[
                   pltpu.VMEM((CH,), jnp.int32),
                   pltpu.VMEM((CH, H), jnp.float32),
                   pltpu.SemaphoreType.DMA,
               ])
    def gather_kernel(table_hbm, idx_hbm, out_hbm, idx_v, rows_v, sem):
        wid = jax.lax.axis_index("s") * NC + jax.lax.axis_index("c")
        base = wid * B_PER_W

        @pl.loop(0, NCH)
        def _(c):
            start = base + c * CH
            pltpu.sync_copy(idx_hbm.at[pl.ds(start, CH)], idx_v)
            pltpu.async_copy(table_hbm.at[idx_v], rows_v, sem).wait()
            pltpu.sync_copy(rows_v, out_hbm.at[pl.ds(start, CH)])

    return gather_kernel(word_table, flat_ids.astype(jnp.int32))


def _tc_finish(emb, pos_emb, vectors, input_pos, ln_gamma, ln_beta):
    def body(pos_idx_ref, emb_ref, pose_ref, vec_ref, gamma_ref, beta_ref,
             out_ref):
        lblk = pl.program_id(0)
        b = pl.program_id(1)
        x = emb_ref[0]
        row = pos_idx_ref[b] - lblk * BL
        rows = jax.lax.broadcasted_iota(jnp.int32, (BL, 1), 0)
        v = vec_ref[pl.ds(b, 1), :]
        x = jnp.where(rows == row, v, x)
        x = x + pose_ref[...]
        mean = jnp.mean(x, axis=1, keepdims=True)
        xc = x - mean
        var = jnp.mean(xc * xc, axis=1, keepdims=True)
        xhat = xc * jax.lax.rsqrt(var + EPS)
        out_ref[0] = xhat * gamma_ref[...] + beta_ref[...]

    grid = (L // BL, B)
    return pl.pallas_call(
        body,
        grid=grid,
        in_specs=[
            pl.BlockSpec(memory_space=pltpu.SMEM),
            pl.BlockSpec((1, BL, H), lambda l, b: (b, l, 0)),
            pl.BlockSpec((BL, H), lambda l, b: (l, 0)),
            pl.BlockSpec(memory_space=pltpu.VMEM),
            pl.BlockSpec((1, H), lambda l, b: (0, 0)),
            pl.BlockSpec((1, H), lambda l, b: (0, 0)),
        ],
        out_specs=pl.BlockSpec((1, BL, H), lambda l, b: (b, l, 0)),
        out_shape=jax.ShapeDtypeStruct((B, L, H), jnp.float32),
        compiler_params=pltpu.CompilerParams(
            dimension_semantics=("arbitrary", "arbitrary")),
    )(input_pos, emb, pos_emb, vectors, ln_gamma, ln_beta)


@jax.jit
def kernel(input_ids, input_pos, vectors, word_table, pos_table, ln_gamma,
           ln_beta):
    flat_ids = input_ids.reshape(-1)
    emb = _sc_gather(word_table, flat_ids).reshape(B, L, H)
    pos_emb = jax.lax.slice(pos_table, (PAD + 1, 0), (PAD + 1 + L, H))
    return _tc_finish(emb, pos_emb, vectors.astype(jnp.float32),
                      input_pos.astype(jnp.int32),
                      ln_gamma.reshape(1, H), ln_beta.reshape(1, H))


# speedup vs baseline: 1.4486x; 1.1251x over previous
import jax
import jax.numpy as jnp
from jax.experimental import pallas as pl
from jax.experimental.pallas import tpu as pltpu
from jax.experimental.pallas import tpu_sc as plsc

B, L, H, V = 16, 2048, 1024, 32000
PAD = 0
EPS = 1e-12

N_TOK = B * L
NC, NS = 2, 16
NW = NC * NS
B_PER_W = N_TOK // NW
CH = 32
NCH = B_PER_W // CH
BL = 256


def _sc_gather(word_table, flat_ids):
    mesh = plsc.VectorSubcoreMesh(core_axis_name="c", subcore_axis_name="s")

    @pl.kernel(out_type=jax.ShapeDtypeStruct((N_TOK, H), word_table.dtype),
               mesh=mesh,
               scratch_types=[
                   pltpu.VMEM((B_PER_W,), jnp.int32),
                   pltpu.VMEM((CH, H), jnp.float32),
                   pltpu.VMEM((CH, H), jnp.float32),
                   pltpu.SemaphoreType.DMA,
                   pltpu.SemaphoreType.DMA,
                   pltpu.SemaphoreType.DMA,
                   pltpu.SemaphoreType.DMA,
               ])
    def gather_kernel(table_hbm, idx_hbm, out_hbm, idx_v, rows0, rows1,
                      g0, g1, w0, w1):
        wid = jax.lax.axis_index("s") * NC + jax.lax.axis_index("c")
        base = wid * B_PER_W
        rows = (rows0, rows1)
        gsem = (g0, g1)
        wsem = (w0, w1)

        pltpu.sync_copy(idx_hbm.at[pl.ds(base, B_PER_W)], idx_v)

        gh = [None, None]
        wh = [None, None]
        for c in range(NCH):
            slot = c % 2
            if wh[slot] is not None:
                wh[slot].wait()
            h = pltpu.make_async_copy(
                table_hbm.at[idx_v.at[pl.ds(c * CH, CH)]], rows[slot],
                gsem[slot])
            h.start()
            gh[slot] = h
            if c >= 1:
                pslot = 1 - slot
                gh[pslot].wait()
                h = pltpu.make_async_copy(
                    rows[pslot], out_hbm.at[pl.ds(base + (c - 1) * CH, CH)],
                    wsem[pslot])
                h.start()
                wh[pslot] = h
        last = (NCH - 1) % 2
        gh[last].wait()
        h = pltpu.make_async_copy(
            rows[last], out_hbm.at[pl.ds(base + (NCH - 1) * CH, CH)],
            wsem[last])
        h.start()
        wh[last] = h
        wh[0].wait()
        wh[1].wait()

    return gather_kernel(word_table, flat_ids.astype(jnp.int32))


def _tc_finish(emb, pos_emb, vectors, input_pos, ln_gamma, ln_beta):
    def body(pos_idx_ref, emb_ref, pose_ref, vec_ref, gamma_ref, beta_ref,
             out_ref):
        lblk = pl.program_id(0)
        b = pl.program_id(1)
        x = emb_ref[0]
        row = pos_idx_ref[b] - lblk * BL
        rows = jax.lax.broadcasted_iota(jnp.int32, (BL, 1), 0)
        v = vec_ref[pl.ds(b, 1), :]
        x = jnp.where(rows == row, v, x)
        x = x + pose_ref[...]
        mean = jnp.mean(x, axis=1, keepdims=True)
        xc = x - mean
        var = jnp.mean(xc * xc, axis=1, keepdims=True)
        xhat = xc * jax.lax.rsqrt(var + EPS)
        out_ref[0] = xhat * gamma_ref[...] + beta_ref[...]

    grid = (L // BL, B)
    return pl.pallas_call(
        body,
        grid=grid,
        in_specs=[
            pl.BlockSpec(memory_space=pltpu.SMEM),
            pl.BlockSpec((1, BL, H), lambda l, b: (b, l, 0)),
            pl.BlockSpec((BL, H), lambda l, b: (l, 0)),
            pl.BlockSpec(memory_space=pltpu.VMEM),
            pl.BlockSpec((1, H), lambda l, b: (0, 0)),
            pl.BlockSpec((1, H), lambda l, b: (0, 0)),
        ],
        out_specs=pl.BlockSpec((1, BL, H), lambda l, b: (b, l, 0)),
        out_shape=jax.ShapeDtypeStruct((B, L, H), jnp.float32),
        compiler_params=pltpu.CompilerParams(
            dimension_semantics=("arbitrary", "arbitrary")),
    )(input_pos, emb, pos_emb, vectors, ln_gamma, ln_beta)


@jax.jit
def kernel(input_ids, input_pos, vectors, word_table, pos_table, ln_gamma,
           ln_beta):
    flat_ids = input_ids.reshape(-1)
    emb = _sc_gather(word_table, flat_ids).reshape(B, L, H)
    pos_emb = jax.lax.slice(pos_table, (PAD + 1, 0), (PAD + 1 + L, H))
    return _tc_finish(emb, pos_emb, vectors.astype(jnp.float32),
                      input_pos.astype(jnp.int32),
                      ln_gamma.reshape(1, H), ln_beta.reshape(1, H))


# speedup vs baseline: 1.5121x; 1.0438x over previous
import jax
import jax.numpy as jnp
from jax.experimental import pallas as pl
from jax.experimental.pallas import tpu as pltpu
from jax.experimental.pallas import tpu_sc as plsc

B, L, H, V = 16, 2048, 1024, 32000
PAD = 0
EPS = 1e-12

NC, NS = 2, 16
NW = NC * NS
NCHUNK = 4
BC = B // NCHUNK
TOK_C = BC * L
B_PER_W = TOK_C // NW
CH = 32
NCH = B_PER_W // CH
BL = 256


def _sc_gather(word_table, flat_ids):
    mesh = plsc.VectorSubcoreMesh(core_axis_name="c", subcore_axis_name="s")

    @pl.kernel(out_type=jax.ShapeDtypeStruct((TOK_C, H), word_table.dtype),
               mesh=mesh,
               scratch_types=[
                   pltpu.VMEM((B_PER_W,), jnp.int32),
                   pltpu.VMEM((CH, H), jnp.float32),
                   pltpu.VMEM((CH, H), jnp.float32),
                   pltpu.SemaphoreType.DMA,
                   pltpu.SemaphoreType.DMA,
                   pltpu.SemaphoreType.DMA,
                   pltpu.SemaphoreType.DMA,
               ])
    def gather_kernel(table_hbm, idx_hbm, out_hbm, idx_v, rows0, rows1,
                      g0, g1, w0, w1):
        wid = jax.lax.axis_index("s") * NC + jax.lax.axis_index("c")
        base = wid * B_PER_W
        rows = (rows0, rows1)
        gsem = (g0, g1)
        wsem = (w0, w1)

        pltpu.sync_copy(idx_hbm.at[pl.ds(base, B_PER_W)], idx_v)

        gh = [None, None]
        wh = [None, None]
        for c in range(NCH):
            slot = c % 2
            if wh[slot] is not None:
                wh[slot].wait()
            h = pltpu.make_async_copy(
                table_hbm.at[idx_v.at[pl.ds(c * CH, CH)]], rows[slot],
                gsem[slot])
            h.start()
            gh[slot] = h
            if c >= 1:
                pslot = 1 - slot
                gh[pslot].wait()
                h = pltpu.make_async_copy(
                    rows[pslot], out_hbm.at[pl.ds(base + (c - 1) * CH, CH)],
                    wsem[pslot])
                h.start()
                wh[pslot] = h
        last = (NCH - 1) % 2
        gh[last].wait()
        h = pltpu.make_async_copy(
            rows[last], out_hbm.at[pl.ds(base + (NCH - 1) * CH, CH)],
            wsem[last])
        h.start()
        wh[last] = h
        wh[0].wait()
        wh[1].wait()

    return gather_kernel(word_table, flat_ids)


def _tc_chunk(k, emb, pos_emb, vectors, input_pos, gamma, beta, prev):
    def body(pos_idx_ref, emb_ref, pose_ref, vec_ref, gamma_ref, beta_ref,
             *rest):
        out_ref = rest[-1]
        lblk = pl.program_id(0)
        b = pl.program_id(1)
        x = emb_ref[0]
        row = pos_idx_ref[k * BC + b] - lblk * BL
        rows = jax.lax.broadcasted_iota(jnp.int32, (BL, 1), 0)
        v = vec_ref[pl.ds(k * BC + b, 1), :]
        x = jnp.where(rows == row, v, x)
        x = x + pose_ref[...]
        mean = jnp.mean(x, axis=1, keepdims=True)
        xc = x - mean
        var = jnp.mean(xc * xc, axis=1, keepdims=True)
        xhat = xc * jax.lax.rsqrt(var + EPS)
        out_ref[0] = xhat * gamma_ref[...] + beta_ref[...]

    in_specs = [
        pl.BlockSpec(memory_space=pltpu.SMEM),
        pl.BlockSpec((1, BL, H), lambda l, b: (b, l, 0)),
        pl.BlockSpec((BL, H), lambda l, b: (l, 0)),
        pl.BlockSpec(memory_space=pltpu.VMEM),
        pl.BlockSpec((1, H), lambda l, b: (0, 0)),
        pl.BlockSpec((1, H), lambda l, b: (0, 0)),
    ]
    args = [input_pos, emb, pos_emb, vectors, gamma, beta]
    kwargs = {}
    if prev is not None:
        in_specs.append(pl.BlockSpec(memory_space=pl.ANY))
        args.append(prev)
        kwargs["input_output_aliases"] = {6: 0}

    return pl.pallas_call(
        body,
        grid=(L // BL, BC),
        in_specs=in_specs,
        out_specs=pl.BlockSpec((1, BL, H),
                               lambda l, b: (k * BC + b, l, 0)),
        out_shape=jax.ShapeDtypeStruct((B, L, H), jnp.float32),
        compiler_params=pltpu.CompilerParams(
            dimension_semantics=("arbitrary", "arbitrary")),
        **kwargs,
    )(*args)


@jax.jit
def kernel(input_ids, input_pos, vectors, word_table, pos_table, ln_gamma,
           ln_beta):
    flat_ids = input_ids.reshape(-1).astype(jnp.int32)
    pos_emb = jax.lax.slice(pos_table, (PAD + 1, 0), (PAD + 1 + L, H))
    input_pos = input_pos.astype(jnp.int32)
    vectors = vectors.astype(jnp.float32)
    gamma = ln_gamma.reshape(1, H)
    beta = ln_beta.reshape(1, H)

    embs = [
        _sc_gather(word_table,
                   jax.lax.slice(flat_ids, (k * TOK_C,), ((k + 1) * TOK_C,)))
        for k in range(NCHUNK)
    ]
    out = None
    for k in range(NCHUNK):
        emb = embs[k].reshape(BC, L, H)
        out = _tc_chunk(k, emb, pos_emb, vectors, input_pos, gamma, beta,
                        out)
    return out


# speedup vs baseline: 1.5196x; 1.0050x over previous
import jax
import jax.numpy as jnp
from jax.experimental import pallas as pl
from jax.experimental.pallas import tpu as pltpu
from jax.experimental.pallas import tpu_sc as plsc

B, L, H, V = 16, 2048, 1024, 32000
PAD = 0
EPS = 1e-12

NC, NS = 2, 16
NW = NC * NS
NCHUNK = 2
BC = B // NCHUNK
TOK_C = BC * L
B_PER_W = TOK_C // NW
CH = 32
NCH = B_PER_W // CH
BL = 256


def _sc_gather(word_table, flat_ids):
    mesh = plsc.VectorSubcoreMesh(core_axis_name="c", subcore_axis_name="s")

    @pl.kernel(out_type=jax.ShapeDtypeStruct((TOK_C, H), word_table.dtype),
               mesh=mesh,
               scratch_types=[
                   pltpu.VMEM((B_PER_W,), jnp.int32),
                   pltpu.VMEM((CH, H), jnp.float32),
                   pltpu.VMEM((CH, H), jnp.float32),
                   pltpu.SemaphoreType.DMA,
                   pltpu.SemaphoreType.DMA,
                   pltpu.SemaphoreType.DMA,
                   pltpu.SemaphoreType.DMA,
               ])
    def gather_kernel(table_hbm, idx_hbm, out_hbm, idx_v, rows0, rows1,
                      g0, g1, w0, w1):
        wid = jax.lax.axis_index("s") * NC + jax.lax.axis_index("c")
        base = wid * B_PER_W
        rows = (rows0, rows1)
        gsem = (g0, g1)
        wsem = (w0, w1)

        pltpu.sync_copy(idx_hbm.at[pl.ds(base, B_PER_W)], idx_v)

        gh = [None, None]
        wh = [None, None]
        for c in range(NCH):
            slot = c % 2
            if wh[slot] is not None:
                wh[slot].wait()
            h = pltpu.make_async_copy(
                table_hbm.at[idx_v.at[pl.ds(c * CH, CH)]], rows[slot],
                gsem[slot])
            h.start()
            gh[slot] = h
            if c >= 1:
                pslot = 1 - slot
                gh[pslot].wait()
                h = pltpu.make_async_copy(
                    rows[pslot], out_hbm.at[pl.ds(base + (c - 1) * CH, CH)],
                    wsem[pslot])
                h.start()
                wh[pslot] = h
        last = (NCH - 1) % 2
        gh[last].wait()
        h = pltpu.make_async_copy(
            rows[last], out_hbm.at[pl.ds(base + (NCH - 1) * CH, CH)],
            wsem[last])
        h.start()
        wh[last] = h
        wh[0].wait()
        wh[1].wait()

    return gather_kernel(word_table, flat_ids)


def _tc_chunk(k, emb, pos_emb, vectors, input_pos, gamma, beta, prev):
    def body(pos_idx_ref, emb_ref, pose_ref, vec_ref, gamma_ref, beta_ref,
             *rest):
        out_ref = rest[-1]
        lblk = pl.program_id(0)
        b = pl.program_id(1)
        x = emb_ref[0]
        row = pos_idx_ref[k * BC + b] - lblk * BL
        rows = jax.lax.broadcasted_iota(jnp.int32, (BL, 1), 0)
        v = vec_ref[pl.ds(k * BC + b, 1), :]
        x = jnp.where(rows == row, v, x)
        x = x + pose_ref[...]
        mean = jnp.mean(x, axis=1, keepdims=True)
        xc = x - mean
        var = jnp.mean(xc * xc, axis=1, keepdims=True)
        xhat = xc * jax.lax.rsqrt(var + EPS)
        out_ref[0] = xhat * gamma_ref[...] + beta_ref[...]

    in_specs = [
        pl.BlockSpec(memory_space=pltpu.SMEM),
        pl.BlockSpec((1, BL, H), lambda l, b: (b, l, 0)),
        pl.BlockSpec((BL, H), lambda l, b: (l, 0)),
        pl.BlockSpec(memory_space=pltpu.VMEM),
        pl.BlockSpec((1, H), lambda l, b: (0, 0)),
        pl.BlockSpec((1, H), lambda l, b: (0, 0)),
    ]
    args = [input_pos, emb, pos_emb, vectors, gamma, beta]
    kwargs = {}
    if prev is not None:
        in_specs.append(pl.BlockSpec(memory_space=pl.ANY))
        args.append(prev)
        kwargs["input_output_aliases"] = {6: 0}

    return pl.pallas_call(
        body,
        grid=(L // BL, BC),
        in_specs=in_specs,
        out_specs=pl.BlockSpec((1, BL, H),
                               lambda l, b: (k * BC + b, l, 0)),
        out_shape=jax.ShapeDtypeStruct((B, L, H), jnp.float32),
        compiler_params=pltpu.CompilerParams(
            dimension_semantics=("arbitrary", "arbitrary")),
        **kwargs,
    )(*args)


@jax.jit
def kernel(input_ids, input_pos, vectors, word_table, pos_table, ln_gamma,
           ln_beta):
    flat_ids = input_ids.reshape(-1).astype(jnp.int32)
    pos_emb = jax.lax.slice(pos_table, (PAD + 1, 0), (PAD + 1 + L, H))
    input_pos = input_pos.astype(jnp.int32)
    vectors = vectors.astype(jnp.float32)
    gamma = ln_gamma.reshape(1, H)
    beta = ln_beta.reshape(1, H)

    embs = [
        _sc_gather(word_table,
                   jax.lax.slice(flat_ids, (k * TOK_C,), ((k + 1) * TOK_C,)))
        for k in range(NCHUNK)
    ]
    out = None
    for k in range(NCHUNK):
        emb = embs[k].reshape(BC, L, H)
        out = _tc_chunk(k, emb, pos_emb, vectors, input_pos, gamma, beta,
                        out)
    return out


# speedup vs baseline: 1.5202x; 1.0004x over previous
import jax
import jax.numpy as jnp
from jax.experimental import pallas as pl
from jax.experimental.pallas import tpu as pltpu
from jax.experimental.pallas import tpu_sc as plsc

B, L, H, V = 16, 2048, 1024, 32000
PAD = 0
EPS = 1e-12

NC, NS = 2, 16
NW = NC * NS
NCHUNK = 2
BC = B // NCHUNK
TOK_C = BC * L
B_PER_W = TOK_C // NW
CH = 32
NCH = B_PER_W // CH
BL = 256


def _sc_gather(word_table, flat_ids):
    mesh = plsc.VectorSubcoreMesh(core_axis_name="c", subcore_axis_name="s")

    NBUF = 3

    @pl.kernel(out_type=jax.ShapeDtypeStruct((TOK_C, H), word_table.dtype),
               mesh=mesh,
               scratch_types=(
                   [pltpu.VMEM((B_PER_W,), jnp.int32)]
                   + [pltpu.VMEM((CH, H), jnp.float32)] * NBUF
                   + [pltpu.SemaphoreType.DMA] * (2 * NBUF)
               ))
    def gather_kernel(table_hbm, idx_hbm, out_hbm, idx_v, *scr):
        rows = scr[:NBUF]
        gsem = scr[NBUF:2 * NBUF]
        wsem = scr[2 * NBUF:]
        wid = jax.lax.axis_index("s") * NC + jax.lax.axis_index("c")
        base = wid * B_PER_W

        pltpu.sync_copy(idx_hbm.at[pl.ds(base, B_PER_W)], idx_v)

        gh = [None] * NBUF
        wh = [None] * NBUF
        for c in range(NCH):
            slot = c % NBUF
            if wh[slot] is not None:
                wh[slot].wait()
            h = pltpu.make_async_copy(
                table_hbm.at[idx_v.at[pl.ds(c * CH, CH)]], rows[slot],
                gsem[slot])
            h.start()
            gh[slot] = h
            if c >= 1:
                pslot = (c - 1) % NBUF
                gh[pslot].wait()
                h = pltpu.make_async_copy(
                    rows[pslot], out_hbm.at[pl.ds(base + (c - 1) * CH, CH)],
                    wsem[pslot])
                h.start()
                wh[pslot] = h
        last = (NCH - 1) % NBUF
        gh[last].wait()
        h = pltpu.make_async_copy(
            rows[last], out_hbm.at[pl.ds(base + (NCH - 1) * CH, CH)],
            wsem[last])
        h.start()
        wh[last] = h
        for b in range(NBUF):
            if wh[b] is not None:
                wh[b].wait()

    return gather_kernel(word_table, flat_ids)


def _tc_chunk(k, emb, pos_emb, vectors, input_pos, gamma, beta, prev):
    def body(pos_idx_ref, emb_ref, pose_ref, vec_ref, gamma_ref, beta_ref,
             *rest):
        out_ref = rest[-1]
        lblk = pl.program_id(0)
        b = pl.program_id(1)
        x = emb_ref[0]
        row = pos_idx_ref[k * BC + b] - lblk * BL
        rows = jax.lax.broadcasted_iota(jnp.int32, (BL, 1), 0)
        v = vec_ref[pl.ds(k * BC + b, 1), :]
        x = jnp.where(rows == row, v, x)
        x = x + pose_ref[...]
        mean = jnp.mean(x, axis=1, keepdims=True)
        xc = x - mean
        var = jnp.mean(xc * xc, axis=1, keepdims=True)
        xhat = xc * jax.lax.rsqrt(var + EPS)
        out_ref[0] = xhat * gamma_ref[...] + beta_ref[...]

    in_specs = [
        pl.BlockSpec(memory_space=pltpu.SMEM),
        pl.BlockSpec((1, BL, H), lambda l, b: (b, l, 0)),
        pl.BlockSpec((BL, H), lambda l, b: (l, 0)),
        pl.BlockSpec(memory_space=pltpu.VMEM),
        pl.BlockSpec((1, H), lambda l, b: (0, 0)),
        pl.BlockSpec((1, H), lambda l, b: (0, 0)),
    ]
    args = [input_pos, emb, pos_emb, vectors, gamma, beta]
    kwargs = {}
    if prev is not None:
        in_specs.append(pl.BlockSpec(memory_space=pl.ANY))
        args.append(prev)
        kwargs["input_output_aliases"] = {6: 0}

    return pl.pallas_call(
        body,
        grid=(L // BL, BC),
        in_specs=in_specs,
        out_specs=pl.BlockSpec((1, BL, H),
                               lambda l, b: (k * BC + b, l, 0)),
        out_shape=jax.ShapeDtypeStruct((B, L, H), jnp.float32),
        compiler_params=pltpu.CompilerParams(
            dimension_semantics=("arbitrary", "arbitrary")),
        **kwargs,
    )(*args)


@jax.jit
def kernel(input_ids, input_pos, vectors, word_table, pos_table, ln_gamma,
           ln_beta):
    flat_ids = input_ids.reshape(-1).astype(jnp.int32)
    pos_emb = jax.lax.slice(pos_table, (PAD + 1, 0), (PAD + 1 + L, H))
    input_pos = input_pos.astype(jnp.int32)
    vectors = vectors.astype(jnp.float32)
    gamma = ln_gamma.reshape(1, H)
    beta = ln_beta.reshape(1, H)

    embs = [
        _sc_gather(word_table,
                   jax.lax.slice(flat_ids, (k * TOK_C,), ((k + 1) * TOK_C,)))
        for k in range(NCHUNK)
    ]
    out = None
    for k in range(NCHUNK):
        emb = embs[k].reshape(BC, L, H)
        out = _tc_chunk(k, emb, pos_emb, vectors, input_pos, gamma, beta,
                        out)
    return out
